# baseline (speedup 1.0000x reference)
import functools

import jax
import jax.numpy as jnp
from jax import lax
from jax.experimental import pallas as pl
from jax.experimental.pallas import tpu as pltpu
from jax.experimental.pallas import tpu_sc as plsc

NUM = 1_000_000
DIM = 64
BATCH = 16384
FEAT = 26

NC = 2
NS = 16
NW = NC * NS

BPW = BATCH // NW
GROUP = 128
JPW = BPW // GROUP
NG = FEAT * JPW
RPW = FEAT * BPW
L = 16


def _sc_gather(idx_flat, table_v):
    mesh = plsc.VectorSubcoreMesh(core_axis_name="c", subcore_axis_name="s")

    @functools.partial(
        pl.kernel,
        mesh=mesh,
        out_type=jax.ShapeDtypeStruct((BATCH, FEAT, DIM), jnp.float32),
        scratch_types=[
            pltpu.VMEM((RPW,), jnp.int32),
            pltpu.VMEM((RPW,), jnp.int32),
            pltpu.VMEM((GROUP, 2 * DIM), jnp.float32),
            pltpu.VMEM((GROUP, 2 * DIM), jnp.float32),
            pltpu.VMEM((GROUP, DIM), jnp.float32),
            pltpu.VMEM((GROUP, DIM), jnp.float32),
            pltpu.SemaphoreType.DMA,
            pltpu.SemaphoreType.DMA,
            pltpu.SemaphoreType.DMA,
            pltpu.SemaphoreType.DMA,
        ],
    )
    def k(idx_hbm, w_hbm, out_hbm, idx_v, half_v,
          gb_a, gb_b, cb_a, cb_b, gs_a, gs_b, ws_a, ws_b):
        wid = lax.axis_index("s") * NC + lax.axis_index("c")
        b0 = wid * BPW

        for f in range(FEAT):
            pltpu.sync_copy(idx_hbm.at[pl.ds(f * BATCH + b0, BPW)],
                            idx_v.at[pl.ds(f * BPW, BPW)])

        def shr(i, carry):
            half_v[pl.ds(i * L, L)] = lax.shift_right_logical(
                idx_v[pl.ds(i * L, L)], 1)
            return carry

        lax.fori_loop(0, RPW // L, shr, 0)

        gbufs = (gb_a, gb_b)
        cbufs = (cb_a, cb_b)
        gsems = (gs_a, gs_b)
        wsems = (ws_a, ws_b)

        def idx_slice(g):
            return half_v.at[pl.ds(g * GROUP, GROUP)]

        def fire_gather(g, p):
            pltpu.async_copy(w_hbm.at[idx_slice(g)], gbufs[p], gsems[p])

        def wait_gather(g, p):
            pltpu.make_async_copy(w_hbm.at[idx_slice(g)], gbufs[p],
                                  gsems[p]).wait()

        def out_slice(g):
            f = lax.shift_right_logical(g, 2)
            j = lax.bitwise_and(g, 3)
            return out_hbm.at[pl.ds(b0 + j * GROUP, GROUP), f]

        def fire_write(g, p):
            pltpu.async_copy(cbufs[p], out_slice(g), wsems[p])

        def wait_write(g, p):
            pltpu.make_async_copy(cbufs[p], out_slice(g), wsems[p]).wait()

        def compact(g, p):
            gb = gbufs[p]
            cb = cbufs[p]

            def stripe(s, carry):
                base = g * GROUP + s * L
                hv = lax.bitwise_and(idx_v[pl.ds(base, L)], 1) * DIM
                for rr in range(L):
                    r = s * L + rr
                    h = hv[rr]
                    for kk in range(DIM // L):
                        cb[r, pl.ds(kk * L, L)] = gb[r, pl.ds(h + kk * L, L)]
                return carry

            lax.fori_loop(0, GROUP // L, stripe, 0)

        fire_gather(0, 0)
        fire_gather(1, 1)

        def pair(t, carry):
            for p in range(2):
                g = 2 * t + p
                wait_gather(g, p)

                @pl.when(g >= 2)
                def _():
                    wait_write(g - 2, p)

                compact(g, p)

                @pl.when(g + 2 < NG)
                def _():
                    fire_gather(g + 2, p)

                fire_write(g, p)
            return carry

        lax.fori_loop(0, NG // 2, pair, 0)
        wait_write(NG - 2, 0)
        wait_write(NG - 1, 1)

    return k(idx_flat, table_v)


def kernel(input, W):
    idx_flat = jnp.transpose(input.astype(jnp.int32)).reshape(-1)
    table_v = jnp.reshape(W, (NUM // 2, 2 * DIM))
    return _sc_gather(idx_flat, table_v)


# speedup vs baseline: 1.3553x; 1.3553x over previous
import functools

import jax
import jax.numpy as jnp
from jax import lax
from jax.experimental import pallas as pl
from jax.experimental.pallas import tpu as pltpu
from jax.experimental.pallas import tpu_sc as plsc

NUM = 1_000_000
DIM = 64
BATCH = 16384
FEAT = 26

NC = 2
NS = 16
NW = NC * NS

BPW = BATCH // NW
GROUP = 128
NG = FEAT * (BPW // GROUP)
RPW = FEAT * BPW
L = 16

SPLIT = 512_000
CB = 2048
TGRID = SPLIT // CB


def _tc_pack(w_t):
    def body(in1, in2, out):
        out[...] = jnp.concatenate(
            [jnp.transpose(in1[...]), jnp.transpose(in2[...])], axis=1)

    return pl.pallas_call(
        body,
        grid=(TGRID,),
        in_specs=[
            pl.BlockSpec((DIM, CB), lambda i: (0, i)),
            pl.BlockSpec(
                (DIM, CB),
                lambda i: (0, jnp.minimum(i + TGRID, NUM // CB))),
        ],
        out_specs=pl.BlockSpec((CB, 2 * DIM), lambda i: (i, 0)),
        out_shape=jax.ShapeDtypeStruct((SPLIT, 2 * DIM), jnp.float32),
    )(w_t, w_t)


def _sc_gather(idx_flat, table_v):
    mesh = plsc.VectorSubcoreMesh(core_axis_name="c", subcore_axis_name="s")

    @functools.partial(
        pl.kernel,
        mesh=mesh,
        out_type=jax.ShapeDtypeStruct((BATCH, FEAT, DIM), jnp.float32),
        scratch_types=[
            pltpu.VMEM((RPW + L,), jnp.int32),
            pltpu.VMEM((RPW,), jnp.int32),
            pltpu.VMEM((GROUP, 2 * DIM), jnp.float32),
            pltpu.VMEM((GROUP, 2 * DIM), jnp.float32),
            pltpu.VMEM((GROUP, DIM), jnp.float32),
            pltpu.VMEM((GROUP, DIM), jnp.float32),
            pltpu.SemaphoreType.DMA,
            pltpu.SemaphoreType.DMA,
            pltpu.SemaphoreType.DMA,
            pltpu.SemaphoreType.DMA,
        ],
    )
    def k(idx_hbm, w_hbm, out_hbm, idx_v, row_v,
          gb_a, gb_b, cb_a, cb_b, gs_a, gs_b, ws_a, ws_b):
        wid = lax.axis_index("s") * NC + lax.axis_index("c")
        b0 = wid * BPW

        for f in range(FEAT):
            pltpu.sync_copy(idx_hbm.at[pl.ds(f * BATCH + b0, BPW)],
                            idx_v.at[pl.ds(f * BPW, BPW)])

        def to_row(i, carry):
            v = idx_v[pl.ds(i * L, L)]
            hi = jnp.where(v >= SPLIT, SPLIT, 0)
            row_v[pl.ds(i * L, L)] = v - hi
            return carry

        lax.fori_loop(0, RPW // L, to_row, 0)

        gbufs = (gb_a, gb_b)
        cbufs = (cb_a, cb_b)
        gsems = (gs_a, gs_b)
        wsems = (ws_a, ws_b)

        def fire_gather(g, p):
            pltpu.async_copy(w_hbm.at[row_v.at[pl.ds(g * GROUP, GROUP)]],
                             gbufs[p], gsems[p])

        def wait_gather(g, p):
            pltpu.make_async_copy(w_hbm.at[row_v.at[pl.ds(g * GROUP, GROUP)]],
                                  gbufs[p], gsems[p]).wait()

        def out_slice(g):
            f = lax.shift_right_logical(g, 2)
            j = lax.bitwise_and(g, 3)
            return out_hbm.at[pl.ds(b0 + j * GROUP, GROUP), f]

        def fire_write(g, p):
            pltpu.async_copy(cbufs[p], out_slice(g), wsems[p])

        def wait_write(g, p):
            pltpu.make_async_copy(cbufs[p], out_slice(g), wsems[p]).wait()

        def compact(g, p):
            gb = gbufs[p]
            cb = cbufs[p]

            def stripe(s, carry):
                base = g * GROUP + s * L
                hv = jnp.where(idx_v[pl.ds(base, L)] >= SPLIT, DIM, 0)
                for rr in range(L):
                    r = s * L + rr
                    h = hv[rr]
                    for kk in range(DIM // L):
                        cb[r, pl.ds(kk * L, L)] = gb[r, pl.ds(h + kk * L, L)]
                return carry

            lax.fori_loop(0, GROUP // L, stripe, 0)

        fire_gather(0, 0)
        fire_gather(1, 1)

        def pair(t, carry):
            for p in range(2):
                g = 2 * t + p
                wait_gather(g, p)

                @pl.when(g >= 2)
                def _():
                    wait_write(g - 2, p)

                compact(g, p)

                @pl.when(g + 2 < NG)
                def _():
                    fire_gather(g + 2, p)

                fire_write(g, p)
            return carry

        lax.fori_loop(0, NG // 2, pair, 0)
        wait_write(NG - 2, 0)
        wait_write(NG - 1, 1)

    return k(idx_flat, table_v)


def kernel(input, W):
    idx_flat = jnp.transpose(input.astype(jnp.int32)).reshape(-1)
    table_v = _tc_pack(jnp.transpose(W))
    return _sc_gather(idx_flat, table_v)
